# Initial kernel scaffold; baseline (speedup 1.0000x reference)
import jax
import jax.numpy as jnp
from jax.experimental import pallas as pl


def kernel(x, router_w, fc1_logits, fc2_logits, bank_fc1_w, bank_fc1_b, bank_fc2_w, bank_fc2_b):
    raise NotImplementedError("write your pallas kernel here")


# speedup vs baseline: 20.2429x; 20.2429x over previous
import functools

import jax
import jax.numpy as jnp
from jax.experimental import pallas as pl
from jax.experimental.pallas import tpu as pltpu

D_MODEL = 768
D_FF = 2048
N_EXPERTS = 16
TOP_K_EXPERTS = 2
N_PRIMITIVES = 8
TOP_K_PRIMITIVES = 4
TEMPERATURE = 1.0
T_TOKENS = 128


def _topk_softmax_coeffs(lg):
    E, P = lg.shape
    pidx = jax.lax.broadcasted_iota(jnp.int32, (E, P), 1)
    rem = lg
    sel = jnp.zeros((E, P), jnp.bool_)
    for _ in range(TOP_K_PRIMITIVES):
        mv = jnp.max(rem, axis=-1, keepdims=True)
        mi = jnp.min(jnp.where(rem >= mv, pidx, P), axis=-1, keepdims=True)
        pick = pidx == mi
        sel = jnp.logical_or(sel, pick)
        rem = jnp.where(pick, -jnp.inf, rem)
    zm = jnp.max(jnp.where(sel, lg, -jnp.inf), axis=-1, keepdims=True)
    w = jnp.where(sel, jnp.exp(lg - zm), 0.0)
    return w / jnp.sum(w, axis=-1, keepdims=True)


def _moe_kernel(xf_ref, rwt_ref, l1_ref, l2_ref, b1w_ref, b1b_ref, b2w_ref,
                b2b_ref, y_ref, aux_ref,
                hp_ref, u_ref, oh1_ref, oh2_ref, g12_ref, a_ref, b1e_ref,
                b2e_ref, xb_ref):
    i = pl.program_id(0)
    P = N_PRIMITIVES
    E = N_EXPERTS
    T = T_TOKENS

    @pl.when(i == 0)
    def _prologue():
        xf = xf_ref[...]
        xb = xf.astype(jnp.bfloat16)
        xb_ref[...] = xb
        logits = jnp.dot(xb, rwt_ref[...].astype(jnp.bfloat16),
                         preferred_element_type=jnp.float32)
        m = jnp.max(logits, axis=-1, keepdims=True)
        ex = jnp.exp(logits - m)
        probs = ex / jnp.sum(ex, axis=-1, keepdims=True)
        eidx = jax.lax.broadcasted_iota(jnp.int32, (T, E), 1)
        v1 = jnp.max(probs, axis=-1, keepdims=True)
        i1 = jnp.min(jnp.where(probs >= v1, eidx, E), axis=-1, keepdims=True)
        m1 = eidx == i1
        p2 = jnp.where(m1, -jnp.inf, probs)
        v2 = jnp.max(p2, axis=-1, keepdims=True)
        i2 = jnp.min(jnp.where(p2 >= v2, eidx, E), axis=-1, keepdims=True)
        m2 = eidx == i2
        denom = v1 + v2 + 1e-8
        g1 = v1 / denom
        g2 = v2 / denom
        oh1_ref[...] = m1.astype(jnp.float32)
        oh2_ref[...] = m2.astype(jnp.float32)
        g12_ref[...] = jnp.concatenate(
            [jnp.broadcast_to(g1, (T, 128)), jnp.broadcast_to(g2, (T, 128))],
            axis=-1)
        counts = jnp.sum(m1.astype(jnp.float32) + m2.astype(jnp.float32),
                         axis=0, keepdims=True)
        f = counts / (jnp.sum(counts, axis=-1, keepdims=True) + 1e-8)
        pm = jnp.mean(probs, axis=0, keepdims=True)
        aux_ref[...] = jnp.sum(f * pm, axis=-1, keepdims=True) * E
        a1 = _topk_softmax_coeffs(l1_ref[...] / TEMPERATURE)
        a2 = _topk_softmax_coeffs(l2_ref[...] / TEMPERATURE)
        a_ref[...] = jnp.concatenate([a1, a2], axis=0)
        b1e_ref[...] = jnp.dot(a1, b1b_ref[...],
                               preferred_element_type=jnp.float32)
        b2e_ref[...] = jnp.dot(a2, b2b_ref[...],
                               preferred_element_type=jnp.float32)

    @pl.when(i < P)
    def _phase_a():
        acc = jnp.dot(xb_ref[...], b1w_ref[0].astype(jnp.bfloat16),
                      preferred_element_type=jnp.float32)
        hp_ref[pl.ds(i, 1)] = acc.astype(jnp.bfloat16)[None]

    @pl.when(i == P)
    def _middle():
        a1 = a_ref[0:E, :]
        a2 = a_ref[E:2 * E, :]
        u_acc = [None] * P
        for k in range(TOP_K_EXPERTS):
            oh = oh1_ref[...] if k == 0 else oh2_ref[...]
            gk = g12_ref[:, 128 * k:128 * k + 1]
            ak = jnp.dot(oh, a1, preferred_element_type=jnp.float32)
            b1tok = jnp.dot(oh, b1e_ref[...],
                            preferred_element_type=jnp.float32)
            mix = b1tok
            for q in range(P):
                mix = mix + ak[:, q:q + 1] * hp_ref[q].astype(jnp.float32)
            h = jax.nn.gelu(mix)
            bk = gk * jnp.dot(oh, a2, preferred_element_type=jnp.float32)
            for p in range(P):
                term = bk[:, p:p + 1] * h
                u_acc[p] = term if u_acc[p] is None else u_acc[p] + term
        for p in range(P):
            u_ref[p] = u_acc[p].astype(jnp.bfloat16)
        g = oh1_ref[...] * g12_ref[:, 0:1] + oh2_ref[...] * g12_ref[:, 128:129]
        y_ref[...] = jnp.dot(g, b2e_ref[...], preferred_element_type=jnp.float32)

    @pl.when(i >= P)
    def _phase_c():
        pc = i - P
        u = u_ref[pl.ds(pc, 1)][0]
        y_ref[...] += jnp.dot(u, b2w_ref[0].astype(jnp.bfloat16),
                              preferred_element_type=jnp.float32)


@functools.partial(jax.jit, static_argnums=())
def kernel(x, router_w, fc1_logits, fc2_logits, bank_fc1_w, bank_fc1_b,
           bank_fc2_w, bank_fc2_b):
    Bq, Sq, D = x.shape
    xf = x.reshape(-1, D)
    T = xf.shape[0]
    P = N_PRIMITIVES
    E = N_EXPERTS
    F = D_FF

    grid = (2 * P,)
    y, aux = pl.pallas_call(
        _moe_kernel,
        grid=grid,
        in_specs=[
            pl.BlockSpec((T, D), lambda i: (0, 0)),
            pl.BlockSpec((D, E), lambda i: (0, 0)),
            pl.BlockSpec((E, P), lambda i: (0, 0)),
            pl.BlockSpec((E, P), lambda i: (0, 0)),
            pl.BlockSpec((1, D, F), lambda i: (jnp.minimum(i, P - 1), 0, 0)),
            pl.BlockSpec((P, F), lambda i: (0, 0)),
            pl.BlockSpec((1, F, D), lambda i: (jnp.maximum(i - P, 0), 0, 0)),
            pl.BlockSpec((P, D), lambda i: (0, 0)),
        ],
        out_specs=[
            pl.BlockSpec((T, D), lambda i: (0, 0)),
            pl.BlockSpec((1, 1), lambda i: (0, 0)),
        ],
        out_shape=[
            jax.ShapeDtypeStruct((T, D), jnp.float32),
            jax.ShapeDtypeStruct((1, 1), jnp.float32),
        ],
        scratch_shapes=[
            pltpu.VMEM((P, T, F), jnp.bfloat16),
            pltpu.VMEM((P, T, F), jnp.bfloat16),
            pltpu.VMEM((T, E), jnp.float32),
            pltpu.VMEM((T, E), jnp.float32),
            pltpu.VMEM((T, 256), jnp.float32),
            pltpu.VMEM((2 * E, P), jnp.float32),
            pltpu.VMEM((E, F), jnp.float32),
            pltpu.VMEM((E, D), jnp.float32),
            pltpu.VMEM((T, D), jnp.bfloat16),
        ],
        compiler_params=pltpu.CompilerParams(
            dimension_semantics=("arbitrary",),
        ),
    )(xf, router_w.T, fc1_logits, fc2_logits, bank_fc1_w, bank_fc1_b,
      bank_fc2_w, bank_fc2_b)
    return y.reshape(Bq, Sq, D), aux[0, 0]


# speedup vs baseline: 21.7974x; 1.0768x over previous
import functools

import jax
import jax.numpy as jnp
from jax.experimental import pallas as pl
from jax.experimental.pallas import tpu as pltpu

D_MODEL = 768
D_FF = 2048
N_EXPERTS = 16
TOP_K_EXPERTS = 2
N_PRIMITIVES = 8
TOP_K_PRIMITIVES = 4
TEMPERATURE = 1.0
T_TOKENS = 128
DEPTH = 3


def _topk_softmax_coeffs(lg):
    E, P = lg.shape
    pidx = jax.lax.broadcasted_iota(jnp.int32, (E, P), 1)
    rem = lg
    sel = jnp.zeros((E, P), jnp.bool_)
    for _ in range(TOP_K_PRIMITIVES):
        mv = jnp.max(rem, axis=-1, keepdims=True)
        mi = jnp.min(jnp.where(rem >= mv, pidx, P), axis=-1, keepdims=True)
        pick = pidx == mi
        sel = jnp.logical_or(sel, pick)
        rem = jnp.where(pick, -jnp.inf, rem)
    zm = jnp.max(jnp.where(sel, lg, -jnp.inf), axis=-1, keepdims=True)
    w = jnp.where(sel, jnp.exp(lg - zm), 0.0)
    return w / jnp.sum(w, axis=-1, keepdims=True)


def _moe_kernel(xf_ref, rwt_ref, l1_ref, l2_ref, b1b_ref, b2b_ref,
                b1w_hbm, b2w_hbm,
                y_ref, aux_ref,
                ring1, ring2, hp_ref, u_ref, sem1, sem2):
    P = N_PRIMITIVES
    E = N_EXPERTS
    T = T_TOKENS

    def cp1(p):
        return pltpu.make_async_copy(b1w_hbm.at[p], ring1.at[p % DEPTH],
                                     sem1.at[p])

    def cp2(p):
        return pltpu.make_async_copy(b2w_hbm.at[p], ring2.at[p % DEPTH],
                                     sem2.at[p])

    for j in range(DEPTH):
        cp1(j).start()
    cp2(0).start()

    xf = xf_ref[...]
    xb = xf.astype(jnp.bfloat16)
    logits = jnp.dot(xb, rwt_ref[...].astype(jnp.bfloat16),
                     preferred_element_type=jnp.float32)
    m = jnp.max(logits, axis=-1, keepdims=True)
    ex = jnp.exp(logits - m)
    probs = ex / jnp.sum(ex, axis=-1, keepdims=True)
    eidx = jax.lax.broadcasted_iota(jnp.int32, (T, E), 1)
    v1 = jnp.max(probs, axis=-1, keepdims=True)
    i1 = jnp.min(jnp.where(probs >= v1, eidx, E), axis=-1, keepdims=True)
    m1 = eidx == i1
    p2 = jnp.where(m1, -jnp.inf, probs)
    v2 = jnp.max(p2, axis=-1, keepdims=True)
    i2 = jnp.min(jnp.where(p2 >= v2, eidx, E), axis=-1, keepdims=True)
    m2 = eidx == i2
    denom = v1 + v2 + 1e-8
    g1 = v1 / denom
    g2 = v2 / denom
    oh1 = m1.astype(jnp.float32)
    oh2 = m2.astype(jnp.float32)
    counts = jnp.sum(oh1 + oh2, axis=0, keepdims=True)
    f = counts / (jnp.sum(counts, axis=-1, keepdims=True) + 1e-8)
    pm = jnp.mean(probs, axis=0, keepdims=True)
    aux_ref[...] = jnp.sum(f * pm, axis=-1, keepdims=True) * E
    a1 = _topk_softmax_coeffs(l1_ref[...] / TEMPERATURE)
    a2 = _topk_softmax_coeffs(l2_ref[...] / TEMPERATURE)
    b1e = jnp.dot(a1, b1b_ref[...], preferred_element_type=jnp.float32)
    b2e = jnp.dot(a2, b2b_ref[...], preferred_element_type=jnp.float32)

    for p in range(P):
        cp1(p).wait()
        acc = jnp.dot(xb, ring1[p % DEPTH].astype(jnp.bfloat16),
                      preferred_element_type=jnp.float32)
        hp_ref[p] = acc.astype(jnp.bfloat16)
        if p + DEPTH < P:
            cp1(p + DEPTH).start()
        elif p + DEPTH - P + 1 < DEPTH:
            cp2(p + DEPTH - P + 1).start()

    u_acc = [None] * P
    for k in range(TOP_K_EXPERTS):
        oh = oh1 if k == 0 else oh2
        gk = g1 if k == 0 else g2
        ak = jnp.dot(oh, a1, preferred_element_type=jnp.float32)
        mix = jnp.dot(oh, b1e, preferred_element_type=jnp.float32)
        for q in range(P):
            mix = mix + ak[:, q:q + 1] * hp_ref[q].astype(jnp.float32)
        h = jax.nn.gelu(mix)
        bk = gk * jnp.dot(oh, a2, preferred_element_type=jnp.float32)
        for p in range(P):
            term = bk[:, p:p + 1] * h
            u_acc[p] = term if u_acc[p] is None else u_acc[p] + term
    for p in range(P):
        u_ref[p] = u_acc[p].astype(jnp.bfloat16)
    g = oh1 * g1 + oh2 * g2
    y = jnp.dot(g, b2e, preferred_element_type=jnp.float32)

    for p in range(P):
        cp2(p).wait()
        y = y + jnp.dot(u_ref[p], ring2[p % DEPTH].astype(jnp.bfloat16),
                        preferred_element_type=jnp.float32)
        if p + DEPTH < P:
            cp2(p + DEPTH).start()
    y_ref[...] = y


@jax.jit
def kernel(x, router_w, fc1_logits, fc2_logits, bank_fc1_w, bank_fc1_b,
           bank_fc2_w, bank_fc2_b):
    Bq, Sq, D = x.shape
    xf = x.reshape(-1, D)
    T = xf.shape[0]
    P = N_PRIMITIVES
    E = N_EXPERTS
    F = D_FF

    vmem = lambda: pl.BlockSpec(memory_space=pltpu.MemorySpace.VMEM)
    any_ = lambda: pl.BlockSpec(memory_space=pl.ANY)
    y, aux = pl.pallas_call(
        _moe_kernel,
        in_specs=[vmem(), vmem(), vmem(), vmem(), vmem(), vmem(),
                  any_(), any_()],
        out_specs=[vmem(), vmem()],
        out_shape=[
            jax.ShapeDtypeStruct((T, D), jnp.float32),
            jax.ShapeDtypeStruct((1, 1), jnp.float32),
        ],
        scratch_shapes=[
            pltpu.VMEM((DEPTH, D, F), jnp.float32),
            pltpu.VMEM((DEPTH, F, D), jnp.float32),
            pltpu.VMEM((P, T, F), jnp.bfloat16),
            pltpu.VMEM((P, T, F), jnp.bfloat16),
            pltpu.SemaphoreType.DMA((P,)),
            pltpu.SemaphoreType.DMA((P,)),
        ],
    )(xf, router_w.T, fc1_logits, fc2_logits, bank_fc1_b, bank_fc2_b,
      bank_fc1_w, bank_fc2_w)
    return y.reshape(Bq, Sq, D), aux[0, 0]
